# baseline (speedup 1.0000x reference)
import functools

import jax
import jax.numpy as jnp
from jax import lax
from jax.experimental import pallas as pl
from jax.experimental.pallas import tpu as pltpu
from jax.experimental.pallas import tpu_sc as plsc

VOCAB_SIZE = 7
EMBED_DIM = 128
BATCH = 4096
SEQ = 201

N = BATCH * SEQ
NUM_WORKERS = 32
PER_W = N // NUM_WORKERS
CHUNK = 192
CHUNKS = PER_W // CHUNK
GROUPS = CHUNK // 16
KBLKS = EMBED_DIM // 16


def _sc_lookup(idx_flat, table_flat):
    mesh = plsc.VectorSubcoreMesh(core_axis_name="c", subcore_axis_name="s")

    @functools.partial(
        pl.kernel,
        mesh=mesh,
        out_type=jax.ShapeDtypeStruct((N * EMBED_DIM,), jnp.float32),
        compiler_params=pltpu.CompilerParams(needs_layout_passes=False),
        scratch_types=[
            pltpu.VMEM((VOCAB_SIZE * EMBED_DIM,), jnp.float32),
            pltpu.VMEM((PER_W,), jnp.int32),
            pltpu.VMEM((CHUNK * EMBED_DIM,), jnp.float32),
            pltpu.VMEM((CHUNK * EMBED_DIM,), jnp.float32),
            pltpu.SemaphoreType.DMA,
            pltpu.SemaphoreType.DMA,
        ],
    )
    def k(idx_hbm, table_hbm, out_hbm, table_v, idx_v, rows0, rows1,
          sem0, sem1):
        wid = lax.axis_index("s") * 2 + lax.axis_index("c")
        base = wid * PER_W
        sems = (sem0, sem1)
        rows = (rows0, rows1)

        pltpu.sync_copy(table_hbm, table_v)
        pltpu.sync_copy(idx_hbm.at[pl.ds(base, PER_W)], idx_v)

        lane = lax.iota(jnp.int32, 16)

        lane128 = lane * EMBED_DIM

        def compute_chunk(j, b):
            srcs = []
            dsts = []
            for g in range(GROUPS):
                idx16 = idx_v[pl.ds(j * CHUNK + g * 16, 16)]
                srcs.append(idx16 * EMBED_DIM)
                dsts.append(lane128 + g * (16 * EMBED_DIM))

            @plsc.parallel_loop(0, EMBED_DIM, unroll=8)
            def col_body(c):
                for g in range(GROUPS):
                    vals = plsc.load_gather(table_v, [srcs[g] + c])
                    plsc.store_scatter(rows[b], [dsts[g] + c], vals)

        def outer(i, carry):
            for b in (0, 1):
                j = 2 * i + b

                @pl.when(j >= 2)
                def _drain():
                    pltpu.make_async_copy(
                        rows[b],
                        out_hbm.at[pl.ds(base * EMBED_DIM, CHUNK * EMBED_DIM)],
                        sems[b],
                    ).wait()

                compute_chunk(j, b)
                pltpu.async_copy(
                    rows[b],
                    out_hbm.at[pl.ds((base + j * CHUNK) * EMBED_DIM,
                                     CHUNK * EMBED_DIM)],
                    sems[b],
                )
            return carry

        lax.fori_loop(0, CHUNKS // 2, outer, 0)
        for b in (0, 1):
            pltpu.make_async_copy(
                rows[b],
                out_hbm.at[pl.ds(base * EMBED_DIM, CHUNK * EMBED_DIM)],
                sems[b],
            ).wait()

    return k(idx_flat, table_flat)


def kernel(padded_sequences, table):
    idx_flat = padded_sequences.reshape(N).astype(jnp.int32)
    out = _sc_lookup(idx_flat, table.astype(jnp.float32).reshape(-1))
    return out.reshape(BATCH, SEQ, EMBED_DIM)


# speedup vs baseline: 2.7412x; 2.7412x over previous
import functools

import jax
import jax.numpy as jnp
from jax import lax
from jax.experimental import pallas as pl
from jax.experimental.pallas import tpu as pltpu
from jax.experimental.pallas import tpu_sc as plsc

VOCAB_SIZE = 7
EMBED_DIM = 128
BATCH = 4096
SEQ = 201

N = BATCH * SEQ
NUM_WORKERS = 32
PER_W = N // NUM_WORKERS
CHUNK = 192
CHUNKS = PER_W // CHUNK
GROUPS = CHUNK // 16
KBLKS = EMBED_DIM // 16


def _sc_lookup(idx_flat, table_flat):
    mesh = plsc.VectorSubcoreMesh(core_axis_name="c", subcore_axis_name="s")

    @functools.partial(
        pl.kernel,
        mesh=mesh,
        out_type=jax.ShapeDtypeStruct((N * EMBED_DIM,), jnp.float32),
        compiler_params=pltpu.CompilerParams(needs_layout_passes=False),
        scratch_types=[
            pltpu.VMEM((VOCAB_SIZE * EMBED_DIM,), jnp.float32),
            pltpu.VMEM((PER_W,), jnp.int32),
            pltpu.VMEM((CHUNK * EMBED_DIM,), jnp.float32),
            pltpu.VMEM((CHUNK * EMBED_DIM,), jnp.float32),
            pltpu.SemaphoreType.DMA,
            pltpu.SemaphoreType.DMA,
        ],
    )
    def k(idx_hbm, table_hbm, out_hbm, table_v, idx_v, rows0, rows1,
          sem0, sem1):
        wid = lax.axis_index("s") * 2 + lax.axis_index("c")
        base = wid * PER_W
        sems = (sem0, sem1)
        rows = (rows0, rows1)

        pltpu.sync_copy(table_hbm, table_v)
        pltpu.sync_copy(idx_hbm.at[pl.ds(base, PER_W)], idx_v)

        lane = lax.iota(jnp.int32, 16)

        lane128 = lane * EMBED_DIM

        def compute_chunk(j, b):
            def group_body(g, carry):
                idx16 = idx_v[pl.ds(j * CHUNK + g * 16, 16)]
                src_base = idx16 * EMBED_DIM
                dst_base = lane128 + g * (16 * EMBED_DIM)

                @plsc.parallel_loop(0, 16, unroll=4)
                def rot_body(r):
                    off = (lane + r) & 15
                    s0 = src_base + off
                    d0 = dst_base + off
                    for kk in range(KBLKS):
                        vals = plsc.load_gather(table_v, [s0 + kk * 16])
                        plsc.store_scatter(rows[b], [d0 + kk * 16], vals)

                return carry

            lax.fori_loop(0, GROUPS, group_body, 0)

        def outer(i, carry):
            for b in (0, 1):
                j = 2 * i + b

                @pl.when(j >= 2)
                def _drain():
                    pltpu.make_async_copy(
                        rows[b],
                        out_hbm.at[pl.ds(base * EMBED_DIM, CHUNK * EMBED_DIM)],
                        sems[b],
                    ).wait()

                compute_chunk(j, b)
                pltpu.async_copy(
                    rows[b],
                    out_hbm.at[pl.ds((base + j * CHUNK) * EMBED_DIM,
                                     CHUNK * EMBED_DIM)],
                    sems[b],
                )
            return carry

        lax.fori_loop(0, CHUNKS // 2, outer, 0)
        for b in (0, 1):
            pltpu.make_async_copy(
                rows[b],
                out_hbm.at[pl.ds(base * EMBED_DIM, CHUNK * EMBED_DIM)],
                sems[b],
            ).wait()

    return k(idx_flat, table_flat)


def kernel(padded_sequences, table):
    idx_flat = padded_sequences.reshape(N).astype(jnp.int32)
    out = _sc_lookup(idx_flat, table.astype(jnp.float32).reshape(-1))
    return out.reshape(BATCH, SEQ, EMBED_DIM)
